# baseline (speedup 1.0000x reference)
import functools

import jax
import jax.numpy as jnp
from jax import lax
from jax.experimental import pallas as pl
from jax.experimental.pallas import tpu as pltpu
from jax.experimental.pallas import tpu_sc as plsc

N_NODES = 50000
N_EDGES = 800000
N_GRAPHS = 128
HID = 64
NP = 50176
EP = 802816
CHUNK = 256
ROW_BLK = 1024
GRID = NP // ROW_BLK
JUNK = NP - 1
PER_TILE_A = EP // 32
PER_TILE_B = EP // 16
SUPER = 14
SUPC = SUPER * CHUNK
NSLICE = NP // 16
BN_SCALE = (1.0 + 1e-5) ** -0.5


@functools.lru_cache(maxsize=None)
def _build_sc_kernels():
    mesh = plsc.VectorSubcoreMesh(core_axis_name="c", subcore_axis_name="s")
    cparams = pltpu.CompilerParams(needs_layout_passes=False,
                                   use_tc_tiling_on_sc=False)
    degree = functools.partial(
        pl.kernel, mesh=mesh, compiler_params=cparams,
        out_type=[jax.ShapeDtypeStruct((2 * NP,), jnp.float32)] * 4,
        scratch_types=[
            pltpu.VMEM((3 * CHUNK,), jnp.float32),
            pltpu.VMEM((CHUNK,), jnp.float32),
            pltpu.VMEM((CHUNK,), jnp.int32),
            pltpu.VMEM((NSLICE,), jnp.float32),
            pltpu.VMEM_SHARED((NP,), jnp.float32),
            pltpu.VMEM_SHARED((NP,), jnp.float32),
            pltpu.VMEM_SHARED((NP,), jnp.float32),
            pltpu.VMEM_SHARED((NP,), jnp.float32),
        ],
    )(_sc_degree_body)
    phase_a = functools.partial(
        pl.kernel, mesh=mesh, compiler_params=cparams,
        out_type=[
            jax.ShapeDtypeStruct((EP,), jnp.float32),
            jax.ShapeDtypeStruct((NP,), jnp.float32),
            jax.ShapeDtypeStruct((NP,), jnp.float32),
        ],
        scratch_types=[
            pltpu.VMEM((CHUNK, HID), jnp.float32),
            pltpu.VMEM((CHUNK, HID), jnp.float32),
            pltpu.VMEM((CHUNK, HID), jnp.float32),
            pltpu.VMEM((CHUNK, HID), jnp.float32),
            pltpu.VMEM((SUPC,), jnp.int32),
            pltpu.VMEM((SUPC,), jnp.int32),
            pltpu.VMEM((3 * SUPC,), jnp.float32),
            pltpu.VMEM((SUPC,), jnp.float32),
            pltpu.VMEM((320,), jnp.float32),
            pltpu.VMEM((NSLICE,), jnp.float32),
            pltpu.VMEM_SHARED((NP,), jnp.float32),
            pltpu.SemaphoreType.DMA,
            pltpu.SemaphoreType.DMA,
            pltpu.SemaphoreType.DMA,
            pltpu.SemaphoreType.DMA,
        ],
    )(_sc_phase_a_body)
    phase_b = functools.partial(
        pl.kernel, mesh=mesh, compiler_params=cparams,
        out_type=[
            jax.ShapeDtypeStruct((NP, 32), jnp.float32),
            jax.ShapeDtypeStruct((NP, 32), jnp.float32),
        ],
        scratch_types=[
            pltpu.VMEM((CHUNK, 32), jnp.float32),
            pltpu.VMEM((CHUNK, 32), jnp.float32),
            pltpu.VMEM((SUPC,), jnp.int32),
            pltpu.VMEM((SUPC,), jnp.int32),
            pltpu.VMEM((SUPC,), jnp.float32),
            pltpu.VMEM((112, 32), jnp.float32),
            pltpu.VMEM_SHARED((NP, 32), jnp.float32),
            pltpu.SemaphoreType.DMA,
            pltpu.SemaphoreType.DMA,
        ],
    )(_sc_phase_b_body)
    return degree, phase_a, phase_b


def _sc_degree_body(eat_hbm, dst_hbm, z1_hbm, dg_hbm, s0_hbm, s1_hbm, s2_hbm,
                    ebuf, ones_v, idx_d, dstage, a_dg, a_s0, a_s1, a_s2):
    c = lax.axis_index("c")
    s = lax.axis_index("s")
    pltpu.sync_copy(z1_hbm, dstage)
    for acc in (a_dg, a_s0, a_s1, a_s2):
        pltpu.sync_copy(dstage, acc.at[pl.ds(s * NSLICE, NSLICE)])
    one = jnp.ones((16,), jnp.float32)
    for i in range(CHUNK // 16):
        ones_v[pl.ds(i * 16, 16)] = one
    plsc.subcore_barrier()
    ebase = (c * 16 + s) * PER_TILE_A

    def chunk(ci, carry):
        base = ebase + ci * CHUNK
        pltpu.sync_copy(dst_hbm.at[pl.ds(base, CHUNK)], idx_d)
        for i in range(3):
            pltpu.sync_copy(eat_hbm.at[pl.ds(i * EP + base, CHUNK)],
                            ebuf.at[pl.ds(i * CHUNK, CHUNK)])
        pltpu.sync_copy(ones_v, a_dg.at[idx_d], add=True)
        pltpu.sync_copy(ebuf.at[pl.ds(0, CHUNK)], a_s0.at[idx_d], add=True)
        pltpu.sync_copy(ebuf.at[pl.ds(CHUNK, CHUNK)], a_s1.at[idx_d], add=True)
        pltpu.sync_copy(ebuf.at[pl.ds(2 * CHUNK, CHUNK)], a_s2.at[idx_d],
                        add=True)
        return carry

    lax.fori_loop(0, PER_TILE_A // CHUNK, chunk, 0)
    plsc.subcore_barrier()
    for acc, out in ((a_dg, dg_hbm), (a_s0, s0_hbm), (a_s1, s1_hbm),
                     (a_s2, s2_hbm)):
        pltpu.sync_copy(acc.at[pl.ds(s * NSLICE, NSLICE)], dstage)
        pltpu.sync_copy(dstage, out.at[pl.ds(c * NP + s * NSLICE, NSLICE)])


def _sc_phase_a_body(xl_hbm, xr_hbm, eat_hbm, src_hbm, dst_hbm, pp_hbm,
                     z1_hbm, ae_hbm, den0_hbm, den1_hbm,
                     rl0, rl1, rr0, rr1, sblk, dblk, eablk, aeblk,
                     pv, dstage, den_sh,
                     sl0, sl1, sr0, sr1):
    c = lax.axis_index("c")
    s = lax.axis_index("s")
    pltpu.sync_copy(z1_hbm, dstage)
    pltpu.sync_copy(dstage, den_sh.at[pl.ds(s * NSLICE, NSLICE)])
    pltpu.sync_copy(pp_hbm, pv)
    plsc.subcore_barrier()
    a6 = [pv[pl.ds(k * 16, 16)] for k in range(4)]
    a4 = [pv[pl.ds(64 + k * 16, 16)] for k in range(4)]
    we = [[pv[pl.ds(128 + 64 * i + 16 * k, 16)] for k in range(4)]
          for i in range(3)]
    lane = lax.iota(jnp.int32, 16)
    zi = jnp.zeros((16,), jnp.int32)
    ebase = (c * 16 + s) * PER_TILE_A
    bufs = [(rl0, rr0, sl0, sr0), (rl1, rr1, sl1, sr1)]

    def issue(k, b):
        rl, rr, semL, semR = b
        off = k * CHUNK
        pltpu.async_copy(xl_hbm.at[sblk.at[pl.ds(off, CHUNK)]], rl, semL)
        pltpu.async_copy(xr_hbm.at[dblk.at[pl.ds(off, CHUNK)]], rr, semR)

    def consume(k, b):
        rl, rr, semL, semR = b
        off = k * CHUNK
        pltpu.make_async_copy(xl_hbm.at[sblk.at[pl.ds(off, CHUNK)]],
                              rl, semL).wait()
        pltpu.make_async_copy(xr_hbm.at[dblk.at[pl.ds(off, CHUNK)]],
                              rr, semR).wait()

        def group(gi, gc):
            goff = gi * 16
            alphav = jnp.zeros((16,), jnp.float32)
            for e in range(16):
                r = goff + e
                fe = zi + (off + r)
                b0 = plsc.load_gather(eablk, [fe])
                b1 = plsc.load_gather(eablk, [fe + SUPC])
                b2 = plsc.load_gather(eablk, [fe + 2 * SUPC])
                acc = None
                for k4 in range(4):
                    sl = pl.ds(k4 * 16, 16)
                    t = rl[r, sl] + rr[r, sl]
                    t = t + b0 * we[0][k4] + b1 * we[1][k4] + b2 * we[2][k4]
                    part = t * a6[k4] + jnp.abs(t) * a4[k4]
                    acc = part if acc is None else acc + part
                alpha_s = jnp.sum(acc)
                alphav = jnp.where(lane == e, alpha_s, alphav)
            aeblk[pl.ds(off + goff, 16)] = jnp.exp(alphav)
            return gc

        lax.fori_loop(0, CHUNK // 16, group, 0)

    def superstep(si, carry):
        base = ebase + si * SUPC
        pltpu.sync_copy(src_hbm.at[pl.ds(base, SUPC)], sblk)
        pltpu.sync_copy(dst_hbm.at[pl.ds(base, SUPC)], dblk)
        for i in range(3):
            pltpu.sync_copy(eat_hbm.at[pl.ds(i * EP + base, SUPC)],
                            eablk.at[pl.ds(i * SUPC, SUPC)])
        issue(0, bufs[0])

        def pair(pi, pc):
            issue(2 * pi + 1, bufs[1])
            consume(2 * pi, bufs[0])

            @pl.when(pi < SUPER // 2 - 1)
            def _():
                issue(2 * pi + 2, bufs[0])

            consume(2 * pi + 1, bufs[1])
            return pc

        lax.fori_loop(0, SUPER // 2, pair, 0)
        pltpu.sync_copy(aeblk, ae_hbm.at[pl.ds(base, SUPC)])
        pltpu.sync_copy(aeblk, den_sh.at[dblk], add=True)
        return carry

    lax.fori_loop(0, PER_TILE_A // SUPC, superstep, 0)
    plsc.subcore_barrier()
    pltpu.sync_copy(den_sh.at[pl.ds(s * NSLICE, NSLICE)], dstage)

    @pl.when(c == 0)
    def _():
        pltpu.sync_copy(dstage, den0_hbm.at[pl.ds(s * NSLICE, NSLICE)])

    @pl.when(c == 1)
    def _():
        pltpu.sync_copy(dstage, den1_hbm.at[pl.ds(s * NSLICE, NSLICE)])


def _sc_phase_b_body(xla_hbm, xlb_hbm, src_hbm, dst_hbm, ae_hbm, z32_hbm,
                     oa_hbm, ob_hbm,
                     rh0, rh1, sblk, dblk, aeblk,
                     sbuf, acc_sh, sem0, sem1):
    c = lax.axis_index("c")
    s = lax.axis_index("s")
    pltpu.sync_copy(z32_hbm, sbuf)
    for t in range(28):
        pltpu.sync_copy(sbuf, acc_sh.at[pl.ds(s * NSLICE + t * 112, 112), :])
    plsc.subcore_barrier()
    lane = lax.iota(jnp.int32, 16)
    ebase = s * PER_TILE_B
    bufs = [(rh0, sem0), (rh1, sem1)]

    def issue(k, b):
        rh, sem = b
        off = k * CHUNK

        @pl.when(c == 0)
        def _():
            pltpu.async_copy(xla_hbm.at[sblk.at[pl.ds(off, CHUNK)]], rh, sem)

        @pl.when(c == 1)
        def _():
            pltpu.async_copy(xlb_hbm.at[sblk.at[pl.ds(off, CHUNK)]], rh, sem)

    def consume(k, b):
        rh, sem = b
        off = k * CHUNK
        pltpu.make_async_copy(xla_hbm.at[sblk.at[pl.ds(off, CHUNK)]],
                              rh, sem).wait()

        def group(gi, gc):
            goff = gi * 16
            aev = aeblk[pl.ds(off + goff, 16)]
            for e in range(16):
                r = goff + e
                bv = jnp.sum(jnp.where(lane == e, aev, 0.0))
                rh[r, pl.ds(0, 16)] = rh[r, pl.ds(0, 16)] * bv
                rh[r, pl.ds(16, 16)] = rh[r, pl.ds(16, 16)] * bv
            return gc

        lax.fori_loop(0, CHUNK // 16, group, 0)
        pltpu.sync_copy(rh, acc_sh.at[dblk.at[pl.ds(off, CHUNK)]],
                        add=True)

    def superstep(si, carry):
        base = ebase + si * SUPC
        pltpu.sync_copy(src_hbm.at[pl.ds(base, SUPC)], sblk)
        pltpu.sync_copy(dst_hbm.at[pl.ds(base, SUPC)], dblk)
        pltpu.sync_copy(ae_hbm.at[pl.ds(base, SUPC)], aeblk)
        issue(0, bufs[0])

        def pair(pi, pc):
            issue(2 * pi + 1, bufs[1])
            consume(2 * pi, bufs[0])

            @pl.when(pi < SUPER // 2 - 1)
            def _():
                issue(2 * pi + 2, bufs[0])

            consume(2 * pi + 1, bufs[1])
            return pc

        lax.fori_loop(0, SUPER // 2, pair, 0)
        return carry

    lax.fori_loop(0, PER_TILE_B // SUPC, superstep, 0)
    plsc.subcore_barrier()
    for t in range(28):
        pltpu.sync_copy(acc_sh.at[pl.ds(s * NSLICE + t * 112, 112), :], sbuf)

        @pl.when(c == 0)
        def _():
            pltpu.sync_copy(sbuf,
                            oa_hbm.at[pl.ds(s * NSLICE + t * 112, 112), :])

        @pl.when(c == 1)
        def _():
            pltpu.sync_copy(sbuf,
                            ob_hbm.at[pl.ds(s * NSLICE + t * 112, 112), :])


EDGE_BLK = 4096


def _edge_c_body(ea_ref, we_ref, c_ref):
    c_ref[...] = ea_ref[...] @ we_ref[...]


def _edge_c(ea_p, we):
    return pl.pallas_call(
        _edge_c_body,
        grid=(EP // EDGE_BLK,),
        in_specs=[
            pl.BlockSpec((EDGE_BLK, 3), lambda i: (i, 0)),
            pl.BlockSpec((3, HID), lambda i: (0, 0)),
        ],
        out_specs=pl.BlockSpec((EDGE_BLK, HID), lambda i: (i, 0)),
        out_shape=jax.ShapeDtypeStruct((EP, HID), jnp.float32),
    )(ea_p, we)


def _dense_pre_body(h_ref, dg_ref, s0_ref, s1_ref, s2_ref, wl_ref, bl_ref,
                    wr_ref, br_ref, we_ref, att_ref,
                    xl_ref, xr_ref, xla_ref, xlb_ref, ael_ref):
    h = h_ref[...]
    xl = h @ wl_ref[...] + bl_ref[...]
    xr = h @ wr_ref[...] + br_ref[...]
    xl_ref[...] = xl
    xr_ref[...] = xr
    xla_ref[...] = xl[:, 0:32]
    xlb_ref[...] = xl[:, 32:64]
    la = jnp.concatenate([s0_ref[...], s1_ref[...], s2_ref[...]], axis=1)
    la = la / jnp.maximum(dg_ref[...], 1.0)
    el = xl + xr + la @ we_ref[...]
    el = jnp.where(el >= 0, el, 0.2 * el)
    alpha_l = jnp.sum(el * att_ref[...], axis=1, keepdims=True)
    ael_ref[...] = jnp.exp(alpha_l)


def _dense_pre(h, dg, s0, s1, s2, wl, bl, wr, br, we, att):
    k = h.shape[1]
    n1 = pl.BlockSpec((ROW_BLK, 1), lambda i: (i, 0))
    return pl.pallas_call(
        _dense_pre_body,
        grid=(GRID,),
        in_specs=[
            pl.BlockSpec((ROW_BLK, k), lambda i: (i, 0)),
            n1, n1, n1, n1,
            pl.BlockSpec((k, HID), lambda i: (0, 0)),
            pl.BlockSpec((1, HID), lambda i: (0, 0)),
            pl.BlockSpec((k, HID), lambda i: (0, 0)),
            pl.BlockSpec((1, HID), lambda i: (0, 0)),
            pl.BlockSpec((3, HID), lambda i: (0, 0)),
            pl.BlockSpec((1, HID), lambda i: (0, 0)),
        ],
        out_specs=[
            pl.BlockSpec((ROW_BLK, HID), lambda i: (i, 0)),
            pl.BlockSpec((ROW_BLK, HID), lambda i: (i, 0)),
            pl.BlockSpec((ROW_BLK, 32), lambda i: (i, 0)),
            pl.BlockSpec((ROW_BLK, 32), lambda i: (i, 0)),
            pl.BlockSpec((ROW_BLK, 1), lambda i: (i, 0)),
        ],
        out_shape=[
            jax.ShapeDtypeStruct((NP, HID), jnp.float32),
            jax.ShapeDtypeStruct((NP, HID), jnp.float32),
            jax.ShapeDtypeStruct((NP, 32), jnp.float32),
            jax.ShapeDtypeStruct((NP, 32), jnp.float32),
            jax.ShapeDtypeStruct((NP, 1), jnp.float32),
        ],
    )(h, dg, s0, s1, s2, wl, bl, wr, br, we, att)


def _combine_body(oa_ref, ob_ref, dn_ref, ael_ref, xl_ref, bias_ref,
                  gamma_ref, beta_ref, h_ref):
    ael = ael_ref[...]
    num = jnp.concatenate([oa_ref[...], ob_ref[...]], axis=1) + ael * xl_ref[...]
    den = dn_ref[...] + ael
    out = num / (den + 1e-16) + bias_ref[...]
    out = gamma_ref[...] * out * BN_SCALE + beta_ref[...]
    h_ref[...] = jnp.maximum(out, 0.0)


def _combine(oa, ob, dn, ael, xl, bias, gamma, beta):
    return pl.pallas_call(
        _combine_body,
        grid=(GRID,),
        in_specs=[
            pl.BlockSpec((ROW_BLK, 32), lambda i: (i, 0)),
            pl.BlockSpec((ROW_BLK, 32), lambda i: (i, 0)),
            pl.BlockSpec((ROW_BLK, 1), lambda i: (i, 0)),
            pl.BlockSpec((ROW_BLK, 1), lambda i: (i, 0)),
            pl.BlockSpec((ROW_BLK, HID), lambda i: (i, 0)),
            pl.BlockSpec((1, HID), lambda i: (0, 0)),
            pl.BlockSpec((1, HID), lambda i: (0, 0)),
            pl.BlockSpec((1, HID), lambda i: (0, 0)),
        ],
        out_specs=pl.BlockSpec((ROW_BLK, HID), lambda i: (i, 0)),
        out_shape=jax.ShapeDtypeStruct((NP, HID), jnp.float32),
    )(oa, ob, dn, ael, xl, bias, gamma, beta)


def _head_body(h_ref, b_ref, wj_ref, bj_ref, wo_ref, bo_ref,
               acc_ref, out_ref):
    i = pl.program_id(0)

    @pl.when(i == 0)
    def _init():
        acc_ref[...] = jnp.zeros_like(acc_ref)

    hjk = h_ref[...] @ wj_ref[...] + bj_ref[...]
    oh = (b_ref[...] == lax.broadcasted_iota(
        jnp.int32, (ROW_BLK, N_GRAPHS), 1)).astype(jnp.float32)
    acc_ref[...] += lax.dot_general(
        oh, hjk, (((0,), (0,)), ((), ())),
        preferred_element_type=jnp.float32)

    @pl.when(i == GRID - 1)
    def _fin():
        a = acc_ref[...]
        pooled = a[:, 0:HID] / jnp.maximum(a[:, HID:HID + 1], 1.0)
        out_ref[...] = pooled @ wo_ref[...] + bo_ref[...]


def _headpool(h, batch2, wj2, bj2, wo, bo):
    return pl.pallas_call(
        _head_body,
        grid=(GRID,),
        in_specs=[
            pl.BlockSpec((ROW_BLK, HID), lambda i: (i, 0)),
            pl.BlockSpec((ROW_BLK, 1), lambda i: (i, 0)),
            pl.BlockSpec((HID, 128), lambda i: (0, 0)),
            pl.BlockSpec((1, 128), lambda i: (0, 0)),
            pl.BlockSpec((HID, 1), lambda i: (0, 0)),
            pl.BlockSpec((1, 1), lambda i: (0, 0)),
        ],
        out_specs=[
            pl.BlockSpec((N_GRAPHS, 128), lambda i: (0, 0)),
            pl.BlockSpec((N_GRAPHS, 1), lambda i: (0, 0)),
        ],
        out_shape=[
            jax.ShapeDtypeStruct((N_GRAPHS, 128), jnp.float32),
            jax.ShapeDtypeStruct((N_GRAPHS, 1), jnp.float32),
        ],
    )(h, batch2, wj2, bj2, wo, bo)[1]


def kernel(x, edge_index, edge_attr, batch, params):
    src = edge_index[0]
    dst = edge_index[1]
    padn = EP - N_EDGES
    src_p = jnp.concatenate([src, jnp.full((padn,), JUNK, jnp.int32)])
    dst_p = jnp.concatenate([dst, jnp.full((padn,), JUNK, jnp.int32)])
    zpad = jnp.zeros((padn,), jnp.float32)
    eat = jnp.concatenate([edge_attr[:, 0], zpad, edge_attr[:, 1], zpad,
                           edge_attr[:, 2], zpad])
    z32 = jnp.zeros((112, 32), jnp.float32)
    z1 = jnp.zeros((NSLICE,), jnp.float32)
    x_p = jnp.pad(x, ((0, NP - N_NODES), (0, HID - x.shape[1])))
    batch2 = jnp.pad(batch, (0, NP - N_NODES),
                     constant_values=N_GRAPHS)[:, None]

    sc_degree, sc_phase_a, sc_phase_b = _build_sc_kernels()
    dgp, s0p, s1p, s2p = sc_degree(eat, dst_p, z1)
    dg = (dgp[:NP] + dgp[NP:])[:, None]
    s0 = (s0p[:NP] + s0p[NP:])[:, None]
    s1 = (s1p[:NP] + s1p[NP:])[:, None]
    s2 = (s2p[:NP] + s2p[NP:])[:, None]

    def layer_step(h, p):
        pp = jnp.concatenate([0.6 * p['att'], 0.4 * p['att'],
                              p['We'].reshape(-1)])
        xl, xr, xla, xlb, ael = _dense_pre(
            h, dg, s0, s1, s2, p['Wl'], p['bl'][None, :],
            p['Wr'], p['br'][None, :], p['We'], p['att'][None, :])
        ae, den0, den1 = sc_phase_a(xl, xr, eat, src_p, dst_p, pp, z1)
        oa, ob = sc_phase_b(xla, xlb, src_p, dst_p, ae, z32)
        dn = (den0 + den1)[:, None]
        return _combine(oa, ob, dn, ael, xl, p['bias'][None, :],
                        p['gamma'][None, :], p['beta'][None, :])

    layers = [dict(p) for p in params['layers']]
    kpad = HID - layers[0]['Wl'].shape[0]
    layers[0]['Wl'] = jnp.pad(layers[0]['Wl'], ((0, kpad), (0, 0)))
    layers[0]['Wr'] = jnp.pad(layers[0]['Wr'], ((0, kpad), (0, 0)))
    stacked = jax.tree.map(lambda *xs: jnp.stack(xs), *layers)
    h, _ = lax.scan(lambda carry, p: (layer_step(carry, p), None),
                    x_p, stacked)

    wj2 = jnp.pad(params['W_jk'], ((0, 0), (0, 64)))
    bj2 = jnp.concatenate([params['b_jk'], jnp.ones((1,), jnp.float32),
                           jnp.zeros((63,), jnp.float32)])[None, :]
    return _headpool(h, batch2, wj2, bj2, params['W_out'],
                     params['b_out'][None, :])
